# Initial kernel scaffold; baseline (speedup 1.0000x reference)
import jax
import jax.numpy as jnp
from jax.experimental import pallas as pl


def kernel(feature, centers, position, pm_camid, pm_pid, camid):
    raise NotImplementedError("write your pallas kernel here")


# speedup vs baseline: 29.9089x; 29.9089x over previous
import functools

import jax
import jax.numpy as jnp
from jax import lax
from jax.experimental import pallas as pl
from jax.experimental.pallas import tpu as pltpu
from jax.experimental.pallas import tpu_sc as plsc

_SCALE = 0.02
_K = 10
_BLK = 128

_INTERPRET = False


def _dist_topk_body(n_total, blk, f_ref, c_ref, pos_ref, vals_ref, idx_ref):
    f = f_ref[0]
    c = c_ref[0]
    pos = pos_ref[:, 0:1]

    fsq = jnp.sum(f * f, axis=1, keepdims=True)
    csq = jnp.sum(c * c, axis=1)
    fc = lax.dot_general(f, c, dimension_numbers=(((1,), (1,)), ((), ())),
                         preferred_element_type=jnp.float32)
    dist = fsq + csq[None, :] - 2.0 * fc

    cols = lax.broadcasted_iota(jnp.int32, (blk, n_total), 1)
    inf = jnp.float32(jnp.inf)
    dist = jnp.where(cols == pos, inf, dist)

    y_sum = jnp.sum(jnp.exp(dist * (-_SCALE)), axis=1, keepdims=True)

    lane16 = lax.broadcasted_iota(jnp.int32, (blk, 16), 1)

    def step(k, carry):
        dist, vals_acc, idx_acc = carry
        g = jnp.min(dist, axis=1, keepdims=True)
        colk = jnp.min(jnp.where(dist == g, cols, n_total), axis=1,
                       keepdims=True)
        dist = jnp.where(cols == colk, inf, dist)
        kept = colk - (colk > pos).astype(jnp.int32)
        vals_acc = jnp.where(lane16 == k, g, vals_acc)
        idx_acc = jnp.where(lane16 == k, kept, idx_acc)
        return dist, vals_acc, idx_acc

    init = (dist, jnp.zeros((blk, 16), jnp.float32),
            jnp.zeros((blk, 16), jnp.int32))
    _, vals_acc, idx_acc = lax.fori_loop(0, _K, step, init, unroll=False)

    x_sum = jnp.sum(
        jnp.where(lane16 < _K, jnp.exp(vals_acc * (-_SCALE)), 0.0),
        axis=1, keepdims=True)
    rt = -jnp.log(x_sum) + jnp.log(y_sum)
    vals_ref[0] = jnp.where(lane16 == _K, rt, vals_acc)
    idx_ref[0] = idx_acc


def _tc_dist_topk(feature, centers, position):
    p_dim, b_dim, d_dim = feature.shape
    n_dim = centers.shape[1]
    blk = _BLK
    nb = b_dim // blk
    grid = (p_dim, nb)
    body = functools.partial(_dist_topk_body, n_dim, blk)
    vals, idx = pl.pallas_call(
        body,
        grid=grid,
        in_specs=[
            pl.BlockSpec((1, blk, d_dim), lambda p, i: (p, i, 0)),
            pl.BlockSpec((1, n_dim, d_dim), lambda p, i: (p, 0, 0)),
            pl.BlockSpec((blk, 1), lambda p, i: (i, 0)),
        ],
        out_specs=[
            pl.BlockSpec((1, blk, 16), lambda p, i: (p, i, 0)),
            pl.BlockSpec((1, blk, 16), lambda p, i: (p, i, 0)),
        ],
        out_shape=[
            jax.ShapeDtypeStruct((p_dim, b_dim, 16), jnp.float32),
            jax.ShapeDtypeStruct((p_dim, b_dim, 16), jnp.int32),
        ],
        interpret=_INTERPRET,
    )(feature, centers, position.reshape(b_dim, 1))
    return vals, idx


def _sc_gather(pm_pid, idx_flat):
    n_dim = pm_pid.shape[0]
    tot = idx_flat.shape[0]
    info = plsc.get_sparse_core_info()
    nw = info.num_cores * info.num_subcores
    lanes = info.num_lanes
    chunk = tot // nw
    mesh = plsc.VectorSubcoreMesh(core_axis_name="c", subcore_axis_name="s")

    @functools.partial(
        pl.kernel,
        mesh=mesh,
        out_type=jax.ShapeDtypeStruct((tot,), jnp.int32),
        scratch_types=[
            pltpu.VMEM((n_dim,), jnp.int32),
            pltpu.VMEM((chunk,), jnp.int32),
            pltpu.VMEM((chunk,), jnp.int32),
        ],
        compiler_params=pltpu.CompilerParams(needs_layout_passes=False),
    )
    def gk(pid_hbm, idx_hbm, out_hbm, table_v, idx_v, outs_v):
        wid = lax.axis_index("s") * info.num_cores + lax.axis_index("c")
        base = wid * chunk
        pltpu.sync_copy(pid_hbm, table_v)
        pltpu.sync_copy(idx_hbm.at[pl.ds(base, chunk)], idx_v)

        def body(j, carry):
            iv = idx_v[pl.ds(j * lanes, lanes)]
            outs_v[pl.ds(j * lanes, lanes)] = plsc.load_gather(table_v, [iv])
            return carry

        lax.fori_loop(0, chunk // lanes, body, 0)
        pltpu.sync_copy(outs_v, out_hbm.at[pl.ds(base, chunk)])

    return gk(pm_pid, idx_flat)


def kernel(feature, centers, position, pm_camid, pm_pid, camid):
    p_dim, b_dim, _ = feature.shape
    vals, kidx = _tc_dist_topk(feature, centers, position)
    rt = vals[:, :, _K]
    l_p = jnp.sum(rt, axis=1) / b_dim
    l_p = jnp.where(jnp.isnan(l_p), jnp.zeros_like(l_p), l_p)
    loss = jnp.sum(l_p) / p_dim
    idx_flat = kidx[:, :, :_K].reshape(-1)
    pos_vid = _sc_gather(pm_pid, idx_flat).reshape(p_dim, b_dim, _K)
    return (loss, pos_vid)


# speedup vs baseline: 32.2593x; 1.0786x over previous
import functools

import jax
import jax.numpy as jnp
from jax import lax
from jax.experimental import pallas as pl
from jax.experimental.pallas import tpu as pltpu
from jax.experimental.pallas import tpu_sc as plsc

_SCALE = 0.02
_K = 10
_BLK = 256

_INTERPRET = False


def _dist_topk_body(n_total, blk, f_ref, c_ref, pos_ref, vals_ref, idx_ref,
                    csq_ref):
    i = pl.program_id(1)

    @pl.when(i == 0)
    def _():
        c = c_ref[0]
        csq_ref[0:1, :] = jnp.sum(c * c, axis=1)[None, :]

    f = f_ref[0]
    pos = pos_ref[:, 0:1]

    fsq = jnp.sum(f * f, axis=1, keepdims=True)
    fc = lax.dot_general(f, c_ref[0], dimension_numbers=(((1,), (1,)), ((), ())),
                         preferred_element_type=jnp.float32)
    cols = lax.broadcasted_iota(jnp.int32, (blk, n_total), 1)
    inf = jnp.float32(jnp.inf)
    dist = fsq + csq_ref[0:1, :] - 2.0 * fc
    dist = jnp.where(cols == pos, inf, dist)

    y_sum = jnp.sum(jnp.exp(dist * (-_SCALE)), axis=1, keepdims=True)

    lane16 = lax.broadcasted_iota(jnp.int32, (blk, 16), 1)

    def step(k, carry):
        g, colp, vals_acc, idx_acc = carry
        dup_ok = (dist == g) & (cols > colp)
        stay_a = jnp.min(jnp.where(dup_ok, cols, n_total), axis=1,
                         keepdims=True)
        adv_b = jnp.min(jnp.where(dist > g, dist, inf), axis=1,
                        keepdims=True)
        col_b = jnp.min(jnp.where(dist == adv_b, cols, n_total), axis=1,
                        keepdims=True)
        stay = stay_a < n_total
        g_k = jnp.where(stay, g, adv_b)
        col_k = jnp.where(stay, stay_a, col_b)
        kept = col_k - (col_k > pos).astype(jnp.int32)
        vals_acc = jnp.where(lane16 == k, g_k, vals_acc)
        idx_acc = jnp.where(lane16 == k, kept, idx_acc)
        return g_k, col_k, vals_acc, idx_acc

    init = (jnp.full((blk, 1), -inf, jnp.float32),
            jnp.full((blk, 1), -1, jnp.int32),
            jnp.zeros((blk, 16), jnp.float32),
            jnp.zeros((blk, 16), jnp.int32))
    _, _, vals_acc, idx_acc = lax.fori_loop(0, _K, step, init, unroll=False)

    x_sum = jnp.sum(
        jnp.where(lane16 < _K, jnp.exp(vals_acc * (-_SCALE)), 0.0),
        axis=1, keepdims=True)
    rt = -jnp.log(x_sum) + jnp.log(y_sum)
    vals_ref[0] = jnp.where(lane16 == _K, rt, vals_acc)
    idx_ref[0] = idx_acc


def _tc_dist_topk(feature, centers, position):
    p_dim, b_dim, d_dim = feature.shape
    n_dim = centers.shape[1]
    blk = _BLK
    nb = b_dim // blk
    grid = (p_dim, nb)
    body = functools.partial(_dist_topk_body, n_dim, blk)
    vals, idx = pl.pallas_call(
        body,
        grid=grid,
        in_specs=[
            pl.BlockSpec((1, blk, d_dim), lambda p, i: (p, i, 0)),
            pl.BlockSpec((1, n_dim, d_dim), lambda p, i: (p, 0, 0)),
            pl.BlockSpec((blk, 1), lambda p, i: (i, 0)),
        ],
        out_specs=[
            pl.BlockSpec((1, blk, 16), lambda p, i: (p, i, 0)),
            pl.BlockSpec((1, blk, 16), lambda p, i: (p, i, 0)),
        ],
        out_shape=[
            jax.ShapeDtypeStruct((p_dim, b_dim, 16), jnp.float32),
            jax.ShapeDtypeStruct((p_dim, b_dim, 16), jnp.int32),
        ],
        scratch_shapes=[pltpu.VMEM((8, n_dim), jnp.float32)],
        interpret=_INTERPRET,
    )(feature, centers, position.reshape(b_dim, 1))
    return vals, idx


def _sc_gather(pm_pid, idx_flat):
    n_dim = pm_pid.shape[0]
    tot = idx_flat.shape[0]
    info = plsc.get_sparse_core_info()
    nw = info.num_cores * info.num_subcores
    lanes = info.num_lanes
    chunk = tot // nw
    mesh = plsc.VectorSubcoreMesh(core_axis_name="c", subcore_axis_name="s")

    @functools.partial(
        pl.kernel,
        mesh=mesh,
        out_type=jax.ShapeDtypeStruct((tot,), jnp.int32),
        scratch_types=[
            pltpu.VMEM((n_dim,), jnp.int32),
            pltpu.VMEM((chunk,), jnp.int32),
            pltpu.VMEM((chunk,), jnp.int32),
        ],
        compiler_params=pltpu.CompilerParams(needs_layout_passes=False),
    )
    def gk(pid_hbm, idx_hbm, out_hbm, table_v, idx_v, outs_v):
        wid = lax.axis_index("s") * info.num_cores + lax.axis_index("c")
        base = wid * chunk
        pltpu.sync_copy(pid_hbm, table_v)
        pltpu.sync_copy(idx_hbm.at[pl.ds(base, chunk)], idx_v)

        def body(j, carry):
            iv = idx_v[pl.ds(j * lanes, lanes)]
            outs_v[pl.ds(j * lanes, lanes)] = plsc.load_gather(table_v, [iv])
            return carry

        lax.fori_loop(0, chunk // lanes, body, 0)
        pltpu.sync_copy(outs_v, out_hbm.at[pl.ds(base, chunk)])

    return gk(pm_pid, idx_flat)


def kernel(feature, centers, position, pm_camid, pm_pid, camid):
    p_dim, b_dim, _ = feature.shape
    vals, kidx = _tc_dist_topk(feature, centers, position)
    rt = vals[:, :, _K]
    l_p = jnp.sum(rt, axis=1) / b_dim
    l_p = jnp.where(jnp.isnan(l_p), jnp.zeros_like(l_p), l_p)
    loss = jnp.sum(l_p) / p_dim
    idx_flat = kidx[:, :, :_K].reshape(-1)
    pos_vid = _sc_gather(pm_pid, idx_flat).reshape(p_dim, b_dim, _K)
    return (loss, pos_vid)
